# Initial kernel scaffold; baseline (speedup 1.0000x reference)
import jax
import jax.numpy as jnp
from jax.experimental import pallas as pl


def kernel(x, edge_index, pos, W_msg, b_msg, W_aggr, b_aggr):
    raise NotImplementedError("write your pallas kernel here")


# speedup vs baseline: 7.7740x; 7.7740x over previous
import functools

import jax
import jax.numpy as jnp
from jax import lax
from jax.experimental import pallas as pl
from jax.experimental.pallas import tpu as pltpu
from jax.experimental.pallas import tpu_sc as plsc

N = 10000
E = 320000
D = 128
P = 2

NC = 2
NS = 16
NW = NC * NS
F = 144
B = 128
NB = 80
E_PAD = NW * NB * B
N_PAD = 10240
NPT = N_PAD // NS

_sc_mesh = plsc.VectorSubcoreMesh(core_axis_name="c", subcore_axis_name="s")


@functools.partial(
    pl.kernel,
    out_type=jax.ShapeDtypeStruct((NC, N_PAD, F), jnp.float32),
    mesh=_sc_mesh,
    compiler_params=pltpu.CompilerParams(use_tc_tiling_on_sc=False),
    scratch_types=[
        pltpu.VMEM((B,), jnp.int32),
        pltpu.VMEM((B,), jnp.int32),
        pltpu.VMEM((B,), jnp.int32),
        pltpu.VMEM((B,), jnp.int32),
        pltpu.VMEM((B, F), jnp.float32),
        pltpu.VMEM((B, F), jnp.float32),
        pltpu.VMEM_SHARED((N_PAD, F), jnp.float32),
        pltpu.SemaphoreType.DMA,
        pltpu.SemaphoreType.DMA,
    ],
)
def _seg_sum_sc(xa_hbm, srcs_hbm, dsts_hbm, zeros_hbm, out_hbm,
                src0, src1, dst0, dst1, rows0, rows1, acc, sem0, sem1):
    cid = lax.axis_index("c")
    sid = lax.axis_index("s")
    wid = cid * NS + sid

    def load_idx(b, s_buf, d_buf):
        pltpu.sync_copy(srcs_hbm.at[wid, b], s_buf)
        pltpu.sync_copy(dsts_hbm.at[wid, b], d_buf)

    pltpu.sync_copy(zeros_hbm, acc.at[pl.ds(sid * NPT, NPT)])
    plsc.subcore_barrier()

    load_idx(0, src0, dst0)
    load_idx(1, src1, dst1)
    pltpu.async_copy(xa_hbm.at[dst0], rows0, sem0)

    def pair(i, _):
        b = 2 * i
        pltpu.async_copy(xa_hbm.at[dst1], rows1, sem1)
        pltpu.make_async_copy(xa_hbm.at[dst0], rows0, sem0).wait()
        pltpu.sync_copy(rows0, acc.at[src0], add=True)
        load_idx(b + 2, src0, dst0)
        pltpu.async_copy(xa_hbm.at[dst0], rows0, sem0)
        pltpu.make_async_copy(xa_hbm.at[dst1], rows1, sem1).wait()
        pltpu.sync_copy(rows1, acc.at[src1], add=True)
        load_idx(b + 3, src1, dst1)
        return 0

    lax.fori_loop(0, NB // 2 - 1, pair, 0)

    pltpu.async_copy(xa_hbm.at[dst1], rows1, sem1)
    pltpu.make_async_copy(xa_hbm.at[dst0], rows0, sem0).wait()
    pltpu.sync_copy(rows0, acc.at[src0], add=True)
    pltpu.make_async_copy(xa_hbm.at[dst1], rows1, sem1).wait()
    pltpu.sync_copy(rows1, acc.at[src1], add=True)

    plsc.subcore_barrier()
    pltpu.sync_copy(acc.at[pl.ds(sid * NPT, NPT)],
                    out_hbm.at[cid, pl.ds(sid * NPT, NPT)])


def _dense_body(x_ref, pos_ref, g0_ref, g1_ref, w1_ref, w2_ref, wp_ref,
                bm_ref, wa1_ref, wa2_ref, ba_ref, o_ref):
    xb = x_ref[...]
    g = g0_ref[...] + g1_ref[...]
    s = g[:, :D]
    sp = g[:, D:D + 2]
    counts = g[:, D + 2:D + 3]
    posb = pos_ref[:, :P]
    q = sp - counts * posb
    wp = wp_ref[...]
    pterm = q[:, 0:1] * wp[0:1, :] + q[:, 1:2] * wp[1:2, :]
    t = jnp.dot(xb, w1_ref[...], preferred_element_type=jnp.float32) + bm_ref[...]
    sums = counts * t + pterm + jnp.dot(s, w2_ref[...],
                                        preferred_element_type=jnp.float32)
    aggr = sums / jnp.maximum(counts, 1.0)
    o_ref[...] = (jnp.dot(xb, wa1_ref[...], preferred_element_type=jnp.float32)
                  + jnp.dot(aggr, wa2_ref[...], preferred_element_type=jnp.float32)
                  + ba_ref[...])


def _dense_tc(x_pad, pos_pad, g0, g1, w1, w2, wp, bm, wa1, wa2, ba):
    bn = 512
    grid = (N_PAD // bn,)
    row_block = lambda d: pl.BlockSpec((bn, d), lambda i: (i, 0))
    full = lambda a, b: pl.BlockSpec((a, b), lambda i: (0, 0))
    return pl.pallas_call(
        _dense_body,
        grid=grid,
        in_specs=[
            row_block(D), row_block(8), row_block(F), row_block(F),
            full(D, D), full(D, D), full(8, D), full(1, D),
            full(D, D), full(D, D), full(1, D),
        ],
        out_specs=row_block(D),
        out_shape=jax.ShapeDtypeStruct((N_PAD, D), jnp.float32),
    )(x_pad, pos_pad, g0, g1, w1, w2, wp, bm, wa1, wa2, ba)


@jax.jit
def kernel(x, edge_index, pos, W_msg, b_msg, W_aggr, b_aggr):
    xa = jnp.concatenate(
        [x, pos, jnp.ones((N, 1), jnp.float32),
         jnp.zeros((N, F - (D + P + 1)), jnp.float32)], axis=1)
    xa = jnp.pad(xa, ((0, N_PAD - N), (0, 0)))

    src = jnp.pad(edge_index[0], (0, E_PAD - E), constant_values=N)
    dst = jnp.pad(edge_index[1], (0, E_PAD - E))
    srcs = src.reshape(NW, NB, B)
    dsts = dst.reshape(NW, NB, B)
    zeros = jnp.zeros((NPT, F), jnp.float32)

    g = _seg_sum_sc(xa, srcs, dsts, zeros)

    x_pad = jnp.pad(x, ((0, N_PAD - N), (0, 0)))
    pos_pad = jnp.pad(pos, ((0, N_PAD - N), (0, 8 - P)))
    w1 = W_msg[:D]
    w2 = W_msg[D:2 * D]
    wp = jnp.pad(W_msg[2 * D:], ((0, 8 - P), (0, 0)))
    wa1 = W_aggr[:D]
    wa2 = W_aggr[D:]
    out = _dense_tc(x_pad, pos_pad, g[0], g[1], w1, w2, wp,
                    b_msg.reshape(1, D), wa1, wa2, b_aggr.reshape(1, D))
    return out[:N]


# speedup vs baseline: 7.9939x; 1.0283x over previous
import functools

import jax
import jax.numpy as jnp
from jax import lax
from jax.experimental import pallas as pl
from jax.experimental.pallas import tpu as pltpu
from jax.experimental.pallas import tpu_sc as plsc

N = 10000
E = 320000
D = 128
P = 2

NC = 2
NS = 16
NW = NC * NS
F = 144
B = 128
NB = 80
E_PAD = NW * NB * B
N_PAD = 10240
NPT = N_PAD // NS

_sc_mesh = plsc.VectorSubcoreMesh(core_axis_name="c", subcore_axis_name="s")


@functools.partial(
    pl.kernel,
    out_type=jax.ShapeDtypeStruct((NC, N_PAD, F), jnp.float32),
    mesh=_sc_mesh,
    compiler_params=pltpu.CompilerParams(use_tc_tiling_on_sc=False),
    scratch_types=[
        pltpu.VMEM((B,), jnp.int32),
        pltpu.VMEM((B,), jnp.int32),
        pltpu.VMEM((B,), jnp.int32),
        pltpu.VMEM((B,), jnp.int32),
        pltpu.VMEM((B, F), jnp.float32),
        pltpu.VMEM((B, F), jnp.float32),
        pltpu.VMEM_SHARED((N_PAD, F), jnp.float32),
        pltpu.SemaphoreType.DMA,
        pltpu.SemaphoreType.DMA,
    ],
)
def _seg_sum_sc(xa_hbm, srcs_hbm, dsts_hbm, zeros_hbm, out_hbm,
                src0, src1, dst0, dst1, rows0, rows1, acc, sem0, sem1):
    cid = lax.axis_index("c")
    sid = lax.axis_index("s")
    wid = cid * NS + sid

    def load_idx(b, s_buf, d_buf):
        pltpu.sync_copy(srcs_hbm.at[wid, b], s_buf)
        pltpu.sync_copy(dsts_hbm.at[wid, b], d_buf)

    pltpu.sync_copy(zeros_hbm, acc.at[pl.ds(sid * NPT, NPT)])
    plsc.subcore_barrier()

    load_idx(0, src0, dst0)
    load_idx(1, src1, dst1)
    pltpu.async_copy(xa_hbm.at[dst0], rows0, sem0)

    def pair(i, _):
        b = 2 * i
        pltpu.async_copy(xa_hbm.at[dst1], rows1, sem1)
        pltpu.make_async_copy(xa_hbm.at[dst0], rows0, sem0).wait()
        pltpu.sync_copy(rows0, acc.at[src0], add=True)
        load_idx(b + 2, src0, dst0)
        pltpu.async_copy(xa_hbm.at[dst0], rows0, sem0)
        pltpu.make_async_copy(xa_hbm.at[dst1], rows1, sem1).wait()
        pltpu.sync_copy(rows1, acc.at[src1], add=True)
        load_idx(b + 3, src1, dst1)
        return 0

    lax.fori_loop(0, NB // 2 - 1, pair, 0)

    pltpu.async_copy(xa_hbm.at[dst1], rows1, sem1)
    pltpu.make_async_copy(xa_hbm.at[dst0], rows0, sem0).wait()
    pltpu.sync_copy(rows0, acc.at[src0], add=True)
    pltpu.make_async_copy(xa_hbm.at[dst1], rows1, sem1).wait()
    pltpu.sync_copy(rows1, acc.at[src1], add=True)

    plsc.subcore_barrier()
    pltpu.sync_copy(acc.at[pl.ds(sid * NPT, NPT)],
                    out_hbm.at[cid, pl.ds(sid * NPT, NPT)])


def _dense_body(x_ref, pos_ref, g0_ref, g1_ref, w1_ref, w2_ref, wp_ref,
                bm_ref, wa1_ref, wa2_ref, ba_ref, o_ref):
    xb = x_ref[...]
    g = g0_ref[...] + g1_ref[...]
    s = g[:, :D]
    sp = g[:, D:D + 2]
    counts = g[:, D + 2:D + 3]
    posb = pos_ref[:, :P]
    q = sp - counts * posb
    wp = wp_ref[...]
    pterm = q[:, 0:1] * wp[0:1, :] + q[:, 1:2] * wp[1:2, :]
    t = jnp.dot(xb, w1_ref[...], preferred_element_type=jnp.float32) + bm_ref[...]
    sums = counts * t + pterm + jnp.dot(s, w2_ref[...],
                                        preferred_element_type=jnp.float32)
    aggr = sums / jnp.maximum(counts, 1.0)
    o_ref[...] = (jnp.dot(xb, wa1_ref[...], preferred_element_type=jnp.float32)
                  + jnp.dot(aggr, wa2_ref[...], preferred_element_type=jnp.float32)
                  + ba_ref[...])


def _dense_tc(x_pad, pos_pad, g0, g1, w1, w2, wp, bm, wa1, wa2, ba):
    bn = 512
    grid = (N_PAD // bn,)
    row_block = lambda d: pl.BlockSpec((bn, d), lambda i: (i, 0))
    full = lambda a, b: pl.BlockSpec((a, b), lambda i: (0, 0))
    return pl.pallas_call(
        _dense_body,
        grid=grid,
        in_specs=[
            row_block(D), row_block(8), row_block(F), row_block(F),
            full(D, D), full(D, D), full(8, D), full(1, D),
            full(D, D), full(D, D), full(1, D),
        ],
        out_specs=row_block(D),
        out_shape=jax.ShapeDtypeStruct((N_PAD, D), jnp.float32),
    )(x_pad, pos_pad, g0, g1, w1, w2, wp, bm, wa1, wa2, ba)


@jax.jit
def kernel(x, edge_index, pos, W_msg, b_msg, W_aggr, b_aggr):
    xa = jnp.concatenate(
        [x, pos, jnp.ones((N, 1), jnp.float32),
         jnp.zeros((N, F - (D + P + 1)), jnp.float32)], axis=1)
    xa = jnp.pad(xa, ((0, N_PAD - N), (0, 0)))
    xa = jnp.concatenate([xa, xa], axis=0)

    src = jnp.pad(edge_index[0], (0, E_PAD - E), constant_values=N)
    dst = jnp.pad(edge_index[1], (0, E_PAD - E))
    srcs = src.reshape(NW, NB, B)
    dsts = dst.reshape(NW, NB, B)
    core_off = jnp.repeat(jnp.array([0, N_PAD], jnp.int32), NS)
    dsts = dsts + core_off[:, None, None]
    zeros = jnp.zeros((NPT, F), jnp.float32)

    g = _seg_sum_sc(xa, srcs, dsts, zeros)

    x_pad = jnp.pad(x, ((0, N_PAD - N), (0, 0)))
    pos_pad = jnp.pad(pos, ((0, N_PAD - N), (0, 8 - P)))
    w1 = W_msg[:D]
    w2 = W_msg[D:2 * D]
    wp = jnp.pad(W_msg[2 * D:], ((0, 8 - P), (0, 0)))
    wa1 = W_aggr[:D]
    wa2 = W_aggr[D:]
    out = _dense_tc(x_pad, pos_pad, g[0], g[1], w1, w2, wp,
                    b_msg.reshape(1, D), wa1, wa2, b_aggr.reshape(1, D))
    return out[:N]
